# baseline (speedup 1.0000x reference)
import functools

import jax
import jax.numpy as jnp
from jax import lax
from jax.experimental import pallas as pl
from jax.experimental.pallas import tpu as pltpu
from jax.experimental.pallas import tpu_sc as plsc

BATCH = 16384
EMBED = 64
NFEAT = 26
PAIR = 2 * EMBED
VOCAB = 1000000

_T = 8192
_W = 4 * _T
_TGRID = -(-VOCAB // _W)
_HPACK = _TGRID * _T

_NW = 32
_BPW = BATCH // _NW
_CHUNK = 128
_NCH = _BPW // _CHUNK


def _pack2(a16, b16):
    ai = lax.convert_element_type(
        lax.bitcast_convert_type(a16, jnp.uint16), jnp.uint32)
    bi = lax.convert_element_type(
        lax.bitcast_convert_type(b16, jnp.uint16), jnp.uint32)
    return lax.bitcast_convert_type((bi << 16) | ai, jnp.float32)


def _tc_transpose_body(ut_ref, it_ref, u4_ref, i4_ref):
    for src, dst in ((ut_ref, u4_ref), (it_ref, i4_ref)):
        x = src[...].astype(jnp.bfloat16)
        t = [lax.transpose(x[:, k * _T:(k + 1) * _T], (1, 0))
             for k in range(4)]
        dst[...] = jnp.concatenate(
            [_pack2(t[0], t[1]), _pack2(t[2], t[3])], axis=1)


_tc_transpose = pl.pallas_call(
    _tc_transpose_body,
    grid=(_TGRID,),
    in_specs=[
        pl.BlockSpec((EMBED, _W), lambda j: (0, j)),
        pl.BlockSpec((EMBED, _W), lambda j: (0, j)),
    ],
    out_specs=[
        pl.BlockSpec((_T, PAIR), lambda j: (j, 0)),
        pl.BlockSpec((_T, PAIR), lambda j: (j, 0)),
    ],
    out_shape=[
        jax.ShapeDtypeStruct((_HPACK, PAIR), jnp.float32),
        jax.ShapeDtypeStruct((_HPACK, PAIR), jnp.float32),
    ],
    compiler_params=pltpu.CompilerParams(
        vmem_limit_bytes=100 * 1024 * 1024),
)


def _sc_gather_body(uemb, iemb, upidx, ipidx, out_u, out_i, idx_v, rows, sem):
    wid = lax.axis_index("s") * 2 + lax.axis_index("c")
    base = wid * _BPW
    pltpu.sync_copy(upidx.at[wid], idx_v)
    ops = [
        pltpu.async_copy(uemb.at[idx_v.at[j]],
                         rows.at[pl.ds(j * _CHUNK, _CHUNK)], sem)
        for j in range(_NCH)
    ]
    for o in ops:
        o.wait()
    pltpu.sync_copy(rows, out_u.at[pl.ds(base, _BPW)])
    pltpu.sync_copy(ipidx.at[wid], idx_v)
    ops = [
        pltpu.async_copy(iemb.at[idx_v.at[j]],
                         rows.at[pl.ds(j * _CHUNK, _CHUNK)], sem)
        for j in range(_NCH)
    ]
    for o in ops:
        o.wait()
    pltpu.sync_copy(rows, out_i.at[pl.ds(base, _BPW)])


@functools.lru_cache(maxsize=1)
def _make_sc_gather():
    return pl.kernel(
        _sc_gather_body,
        mesh=plsc.VectorSubcoreMesh(core_axis_name="c", subcore_axis_name="s"),
        out_type=[
            jax.ShapeDtypeStruct((BATCH, PAIR), jnp.float32),
            jax.ShapeDtypeStruct((BATCH, PAIR), jnp.float32),
        ],
        scratch_types=[
            pltpu.VMEM((_NCH, _CHUNK), jnp.int32),
            pltpu.VMEM((_BPW, PAIR), jnp.float32),
            pltpu.SemaphoreType.DMA,
        ],
        compiler_params=pltpu.CompilerParams(use_tc_tiling_on_sc=True),
    )


_TC_BLOCK = 4096
_TC_GRID = BATCH // _TC_BLOCK


def _unpack(packed_ref, sel):
    gi = lax.bitcast_convert_type(packed_ref[...], jnp.uint32)
    grp = jnp.where(sel >= 2, gi[:, EMBED:], gi[:, :EMBED])
    bits = jnp.where((sel & 1) == 1,
                     grp & jnp.uint32(0xFFFF0000), grp << 16)
    return lax.bitcast_convert_type(bits, jnp.float32)


def _tc_combine_body(up_ref, ip_ref, usel_ref, isel_ref, uft_ref, ift_ref,
                     wu_ref, bu_ref, wi_ref, bi_ref, out_ref):
    dn = (((0,), (0,)), ((), ()))
    pu = lax.dot_general(uft_ref[...], wu_ref[...], dn,
                         preferred_element_type=jnp.float32) + bu_ref[...]
    pi = lax.dot_general(ift_ref[...], wi_ref[...], dn,
                         preferred_element_type=jnp.float32) + bi_ref[...]
    ug = _unpack(up_ref, usel_ref[...])
    ig = _unpack(ip_ref, isel_ref[...])
    out_ref[...] = jnp.sum((ug + pu) * (ig + pi), axis=1, keepdims=True)


_tc_combine = pl.pallas_call(
    _tc_combine_body,
    grid=(_TC_GRID,),
    in_specs=[
        pl.BlockSpec((_TC_BLOCK, PAIR), lambda i: (i, 0)),
        pl.BlockSpec((_TC_BLOCK, PAIR), lambda i: (i, 0)),
        pl.BlockSpec((_TC_BLOCK, 1), lambda i: (i, 0)),
        pl.BlockSpec((_TC_BLOCK, 1), lambda i: (i, 0)),
        pl.BlockSpec((NFEAT, _TC_BLOCK), lambda i: (0, i)),
        pl.BlockSpec((NFEAT, _TC_BLOCK), lambda i: (0, i)),
        pl.BlockSpec((NFEAT, EMBED), lambda i: (0, 0)),
        pl.BlockSpec((1, EMBED), lambda i: (0, 0)),
        pl.BlockSpec((NFEAT, EMBED), lambda i: (0, 0)),
        pl.BlockSpec((1, EMBED), lambda i: (0, 0)),
    ],
    out_specs=pl.BlockSpec((_TC_BLOCK, 1), lambda i: (i, 0)),
    out_shape=jax.ShapeDtypeStruct((BATCH, 1), jnp.float32),
)


def _pack_coords(idx):
    j = idx // _W
    w = idx - j * _W
    prow = j * _T + lax.rem(w, _T)
    sel = w // _T
    return prow, sel


def kernel(user_indices, item_indices, user_features, item_features,
           user_emb, item_emb, Wu, bu, Wi, bi):
    ui = user_indices.astype(jnp.int32)
    ii = item_indices.astype(jnp.int32)
    uprow, usel = _pack_coords(ui)
    iprow, isel = _pack_coords(ii)
    upidx = uprow.reshape(_NW, _NCH, _CHUNK)
    ipidx = iprow.reshape(_NW, _NCH, _CHUNK)
    u4, i4 = _tc_transpose(user_emb.T, item_emb.T)
    pairs_u, pairs_i = _make_sc_gather()(u4, i4, upidx, ipidx)
    out = _tc_combine(pairs_u, pairs_i,
                      usel.reshape(BATCH, 1), isel.reshape(BATCH, 1),
                      user_features.T, item_features.T,
                      Wu, bu.reshape(1, EMBED), Wi, bi.reshape(1, EMBED))
    return out.reshape(BATCH)


# speedup vs baseline: 1.0044x; 1.0044x over previous
import functools

import jax
import jax.numpy as jnp
from jax import lax
from jax.experimental import pallas as pl
from jax.experimental.pallas import tpu as pltpu
from jax.experimental.pallas import tpu_sc as plsc

BATCH = 16384
EMBED = 64
NFEAT = 26
PAIR = 2 * EMBED
VOCAB = 1000000

_T = 8192
_W = 4 * _T
_TGRID = -(-VOCAB // _W)
_HPACK = _TGRID * _T

_NW = 32
_BPW = BATCH // _NW
_CHUNK = 128
_NCH = _BPW // _CHUNK


def _pack2(a16, b16):
    ai = lax.convert_element_type(
        lax.bitcast_convert_type(a16, jnp.uint16), jnp.uint32)
    bi = lax.convert_element_type(
        lax.bitcast_convert_type(b16, jnp.uint16), jnp.uint32)
    return lax.bitcast_convert_type((bi << 16) | ai, jnp.float32)


def _tc_transpose_body(ut_ref, it_ref, u4_ref, i4_ref):
    for src, dst in ((ut_ref, u4_ref), (it_ref, i4_ref)):
        x = src[...].astype(jnp.bfloat16)
        t = [lax.transpose(x[:, k * _T:(k + 1) * _T], (1, 0))
             for k in range(4)]
        dst[...] = jnp.concatenate(
            [_pack2(t[0], t[1]), _pack2(t[2], t[3])], axis=1)


_tc_transpose = pl.pallas_call(
    _tc_transpose_body,
    grid=(_TGRID,),
    in_specs=[
        pl.BlockSpec((EMBED, _W), lambda j: (0, j)),
        pl.BlockSpec((EMBED, _W), lambda j: (0, j)),
    ],
    out_specs=[
        pl.BlockSpec((_T, PAIR), lambda j: (j, 0)),
        pl.BlockSpec((_T, PAIR), lambda j: (j, 0)),
    ],
    out_shape=[
        jax.ShapeDtypeStruct((_HPACK, PAIR), jnp.float32),
        jax.ShapeDtypeStruct((_HPACK, PAIR), jnp.float32),
    ],
    compiler_params=pltpu.CompilerParams(
        vmem_limit_bytes=100 * 1024 * 1024),
)


def _sc_gather_body(uemb, iemb, upidx, ipidx, out_u, out_i, idx_v, rows, sem):
    wid = lax.axis_index("s") * 2 + lax.axis_index("c")
    base = wid * _BPW
    pltpu.sync_copy(upidx.at[wid], idx_v)
    ops = [
        pltpu.async_copy(uemb.at[idx_v.at[j]],
                         rows.at[pl.ds(j * _CHUNK, _CHUNK)], sem)
        for j in range(_NCH)
    ]
    for o in ops:
        o.wait()
    pltpu.sync_copy(rows, out_u.at[pl.ds(base, _BPW)])
    pltpu.sync_copy(ipidx.at[wid], idx_v)
    ops = [
        pltpu.async_copy(iemb.at[idx_v.at[j]],
                         rows.at[pl.ds(j * _CHUNK, _CHUNK)], sem)
        for j in range(_NCH)
    ]
    for o in ops:
        o.wait()
    pltpu.sync_copy(rows, out_i.at[pl.ds(base, _BPW)])


@functools.lru_cache(maxsize=1)
def _make_sc_gather():
    return pl.kernel(
        _sc_gather_body,
        mesh=plsc.VectorSubcoreMesh(core_axis_name="c", subcore_axis_name="s"),
        out_type=[
            jax.ShapeDtypeStruct((BATCH, PAIR), jnp.float32),
            jax.ShapeDtypeStruct((BATCH, PAIR), jnp.float32),
        ],
        scratch_types=[
            pltpu.VMEM((_NCH, _CHUNK), jnp.int32),
            pltpu.VMEM((_BPW, PAIR), jnp.float32),
            pltpu.SemaphoreType.DMA,
        ],
        compiler_params=pltpu.CompilerParams(use_tc_tiling_on_sc=True),
    )


_TC_BLOCK = 2048
_TC_GRID = BATCH // _TC_BLOCK


def _unpack(packed_ref, sel):
    gi = lax.bitcast_convert_type(packed_ref[...], jnp.uint32)
    grp = jnp.where(sel >= 2, gi[:, EMBED:], gi[:, :EMBED])
    bits = jnp.where((sel & 1) == 1,
                     grp & jnp.uint32(0xFFFF0000), grp << 16)
    return lax.bitcast_convert_type(bits, jnp.float32)


def _tc_combine_body(up_ref, ip_ref, usel_ref, isel_ref, uft_ref, ift_ref,
                     wu_ref, bu_ref, wi_ref, bi_ref, out_ref):
    dn = (((0,), (0,)), ((), ()))
    pu = lax.dot_general(uft_ref[...], wu_ref[...], dn,
                         preferred_element_type=jnp.float32) + bu_ref[...]
    pi = lax.dot_general(ift_ref[...], wi_ref[...], dn,
                         preferred_element_type=jnp.float32) + bi_ref[...]
    ug = _unpack(up_ref, usel_ref[...])
    ig = _unpack(ip_ref, isel_ref[...])
    out_ref[...] = jnp.sum((ug + pu) * (ig + pi), axis=1, keepdims=True)


_tc_combine = pl.pallas_call(
    _tc_combine_body,
    grid=(_TC_GRID,),
    in_specs=[
        pl.BlockSpec((_TC_BLOCK, PAIR), lambda i: (i, 0)),
        pl.BlockSpec((_TC_BLOCK, PAIR), lambda i: (i, 0)),
        pl.BlockSpec((_TC_BLOCK, 1), lambda i: (i, 0)),
        pl.BlockSpec((_TC_BLOCK, 1), lambda i: (i, 0)),
        pl.BlockSpec((NFEAT, _TC_BLOCK), lambda i: (0, i)),
        pl.BlockSpec((NFEAT, _TC_BLOCK), lambda i: (0, i)),
        pl.BlockSpec((NFEAT, EMBED), lambda i: (0, 0)),
        pl.BlockSpec((1, EMBED), lambda i: (0, 0)),
        pl.BlockSpec((NFEAT, EMBED), lambda i: (0, 0)),
        pl.BlockSpec((1, EMBED), lambda i: (0, 0)),
    ],
    out_specs=pl.BlockSpec((_TC_BLOCK, 1), lambda i: (i, 0)),
    out_shape=jax.ShapeDtypeStruct((BATCH, 1), jnp.float32),
)


def _pack_coords(idx):
    j = idx // _W
    w = idx - j * _W
    prow = j * _T + lax.rem(w, _T)
    sel = w // _T
    return prow, sel


def kernel(user_indices, item_indices, user_features, item_features,
           user_emb, item_emb, Wu, bu, Wi, bi):
    ui = user_indices.astype(jnp.int32)
    ii = item_indices.astype(jnp.int32)
    uprow, usel = _pack_coords(ui)
    iprow, isel = _pack_coords(ii)
    upidx = uprow.reshape(_NW, _NCH, _CHUNK)
    ipidx = iprow.reshape(_NW, _NCH, _CHUNK)
    u4, i4 = _tc_transpose(user_emb.T, item_emb.T)
    pairs_u, pairs_i = _make_sc_gather()(u4, i4, upidx, ipidx)
    out = _tc_combine(pairs_u, pairs_i,
                      usel.reshape(BATCH, 1), isel.reshape(BATCH, 1),
                      user_features.T, item_features.T,
                      Wu, bu.reshape(1, EMBED), Wi, bi.reshape(1, EMBED))
    return out.reshape(BATCH)
